# Initial kernel scaffold; baseline (speedup 1.0000x reference)
import functools

import jax
import jax.numpy as jnp
from jax import lax
from jax.experimental import pallas as pl
from jax.experimental.pallas import tpu as pltpu, tpu_sc as plsc

N_SITES = 50000
N_IN = 128
N_OUT = 128
FV = 9

NW = 32
NPAD = 50176
CHUNK = NPAD // NW
SUB = CHUNK // 2
BN = 1024


def _mm_body(feat_ref, w_ref, b_ref, out_ref):
    f = feat_ref[...]
    for k in range(FV):
        out_ref[k] = (
            jnp.dot(f, w_ref[k], preferred_element_type=jnp.float32)
            + b_ref[0] * (1.0 / FV)
        )


def _transform(feat_pad, weight, bias):
    grid = (NPAD // BN,)
    return pl.pallas_call(
        _mm_body,
        grid=grid,
        in_specs=[
            pl.BlockSpec((BN, N_IN), lambda i: (i, 0)),
            pl.BlockSpec((FV, N_IN, N_OUT), lambda i: (0, 0, 0)),
            pl.BlockSpec((1, N_OUT), lambda i: (0, 0)),
        ],
        out_specs=pl.BlockSpec((FV, BN, N_OUT), lambda i: (0, i, 0)),
        out_shape=jax.ShapeDtypeStruct((FV, NPAD, N_OUT), jnp.float32),
    )(feat_pad, weight, bias.reshape(1, N_OUT))


def _sc_body(t_hbm, idx_hbm, out_hbm, idx_v, acc_v, sem):
    c = lax.axis_index("c")
    s = lax.axis_index("s")
    wid = s * 2 + c
    base = wid * CHUNK
    for i in range(CHUNK // SUB):
        off = base + i * SUB
        pltpu.sync_copy(idx_hbm.at[:, pl.ds(off, SUB)], idx_v)
        pltpu.async_copy(t_hbm.at[idx_v.at[0]], acc_v, sem).wait()
        for f in range(1, FV):
            pltpu.async_copy(t_hbm.at[idx_v.at[f]], acc_v, sem, add=True).wait()
        pltpu.sync_copy(acc_v, out_hbm.at[pl.ds(off, SUB)])


_gather_sum = functools.partial(
    pl.kernel,
    out_type=jax.ShapeDtypeStruct((NPAD, N_OUT), jnp.float32),
    mesh=plsc.VectorSubcoreMesh(core_axis_name="c", subcore_axis_name="s"),
    scratch_types=[
        pltpu.VMEM((FV, SUB), jnp.int32),
        pltpu.VMEM((SUB, N_OUT), jnp.float32),
        pltpu.SemaphoreType.DMA,
    ],
)(_sc_body)


@jax.jit
def kernel(features, neighbor_idx, weight, bias):
    feat_pad = jnp.pad(features, ((0, NPAD - N_SITES), (0, 0)))
    t = _transform(feat_pad, weight, bias)
    t_flat = t.reshape(FV * NPAD, N_OUT)
    idx_t = (
        jnp.pad(neighbor_idx, ((0, NPAD - N_SITES), (0, 0))).T
        + (jnp.arange(FV, dtype=jnp.int32) * NPAD)[:, None]
    )
    out = _gather_sum(t_flat, idx_t)
    return out[:N_SITES]


# speedup vs baseline: 4.2098x; 4.2098x over previous
import functools

import jax
import jax.numpy as jnp
from jax import lax
from jax.experimental import pallas as pl
from jax.experimental.pallas import tpu as pltpu, tpu_sc as plsc

N_SITES = 50000
N_IN = 128
N_OUT = 128
FV = 9

NW = 32
NPAD = 50176
CHUNK = NPAD // NW
SUB = CHUNK // 2
BN = 1024


def _mm_body(feat_ref, w_ref, b_ref, out_ref):
    f = feat_ref[...]
    for k in range(FV):
        out_ref[k] = (
            jnp.dot(f, w_ref[k], preferred_element_type=jnp.float32)
            + b_ref[0] * (1.0 / FV)
        )


def _transform(feat_pad, weight, bias):
    grid = (NPAD // BN,)
    return pl.pallas_call(
        _mm_body,
        grid=grid,
        in_specs=[
            pl.BlockSpec((BN, N_IN), lambda i: (i, 0)),
            pl.BlockSpec((FV, N_IN, N_OUT), lambda i: (0, 0, 0)),
            pl.BlockSpec((1, N_OUT), lambda i: (0, 0)),
        ],
        out_specs=pl.BlockSpec((FV, BN, N_OUT), lambda i: (0, i, 0)),
        out_shape=jax.ShapeDtypeStruct((FV, NPAD, N_OUT), jnp.float32),
    )(feat_pad, weight, bias.reshape(1, N_OUT))


def _sc_body(t_hbm, idx_hbm, out_hbm, idx_v, acc_v, sem):
    c = lax.axis_index("c")
    s = lax.axis_index("s")
    wid = s * 2 + c
    base = wid * CHUNK
    for i in range(CHUNK // SUB):
        off = base + i * SUB
        for f in range(FV):
            pltpu.sync_copy(
                idx_hbm.at[pl.ds(f * NPAD + off, SUB)],
                idx_v.at[pl.ds(f * SUB, SUB)],
            )
        pltpu.async_copy(t_hbm.at[idx_v.at[pl.ds(0, SUB)]], acc_v, sem).wait()
        for f in range(1, FV):
            pltpu.async_copy(
                t_hbm.at[idx_v.at[pl.ds(f * SUB, SUB)]], acc_v, sem, add=True
            ).wait()
        pltpu.sync_copy(acc_v, out_hbm.at[pl.ds(off, SUB)])


_gather_sum = functools.partial(
    pl.kernel,
    out_type=jax.ShapeDtypeStruct((NPAD, N_OUT), jnp.float32),
    mesh=plsc.VectorSubcoreMesh(core_axis_name="c", subcore_axis_name="s"),
    scratch_types=[
        pltpu.VMEM((FV * SUB,), jnp.int32),
        pltpu.VMEM((SUB, N_OUT), jnp.float32),
        pltpu.SemaphoreType.DMA,
    ],
)(_sc_body)


@jax.jit
def kernel(features, neighbor_idx, weight, bias):
    feat_pad = jnp.pad(features, ((0, NPAD - N_SITES), (0, 0)))
    t = _transform(feat_pad, weight, bias)
    t_flat = t.reshape(FV * NPAD, N_OUT)
    idx_t = (
        jnp.pad(neighbor_idx, ((0, NPAD - N_SITES), (0, 0))).T
        + (jnp.arange(FV, dtype=jnp.int32) * NPAD)[:, None]
    )
    out = _gather_sum(t_flat, idx_t.reshape(FV * NPAD))
    return out[:N_SITES]


# speedup vs baseline: 6.7977x; 1.6147x over previous
import functools

import jax
import jax.numpy as jnp
from jax import lax
from jax.experimental import pallas as pl
from jax.experimental.pallas import tpu as pltpu, tpu_sc as plsc

N_SITES = 50000
N_IN = 128
N_OUT = 128
FV = 9

NW = 32
CHUNK = 1568
SUB = 784
CHUNK_L = N_SITES - (NW - 1) * CHUNK
SUB_L = CHUNK_L // 2
BN = 1024


def _mm_body(feat_ref, w_ref, b_ref, out_ref):
    f = feat_ref[...]
    for k in range(FV):
        out_ref[k] = (
            jnp.dot(f, w_ref[k], preferred_element_type=jnp.float32)
            + b_ref[0] * (1.0 / FV)
        )


def _transform(features, weight, bias):
    grid = (pl.cdiv(N_SITES, BN),)
    return pl.pallas_call(
        _mm_body,
        grid=grid,
        in_specs=[
            pl.BlockSpec((BN, N_IN), lambda i: (i, 0)),
            pl.BlockSpec((FV, N_IN, N_OUT), lambda i: (0, 0, 0)),
            pl.BlockSpec((1, N_OUT), lambda i: (0, 0)),
        ],
        out_specs=pl.BlockSpec((FV, BN, N_OUT), lambda i: (0, i, 0)),
        out_shape=jax.ShapeDtypeStruct((FV, N_SITES, N_OUT), jnp.float32),
    )(features, weight, bias.reshape(1, N_OUT))


def _work(t_hbm, idx_hbm, out_hbm, idx_v, acc_v, sem, base, chunk, sub):
    for f in range(FV):
        pltpu.sync_copy(
            idx_hbm.at[pl.ds(f * N_SITES + base, chunk)],
            idx_v.at[pl.ds(f * chunk, chunk)],
        )
    for i in range(chunk // sub):
        off = base + i * sub
        acc = acc_v.at[pl.ds(0, sub)]
        pltpu.async_copy(
            t_hbm.at[idx_v.at[pl.ds(i * sub, sub)]], acc, sem
        ).wait()
        for f in range(1, FV):
            pltpu.async_copy(
                t_hbm.at[idx_v.at[pl.ds(f * chunk + i * sub, sub)]],
                acc,
                sem,
                add=True,
            ).wait()
        pltpu.sync_copy(acc, out_hbm.at[pl.ds(off, sub)])


def _sc_body(t_hbm, idx_hbm, out_hbm, idx_v, acc_v, sem):
    c = lax.axis_index("c")
    s = lax.axis_index("s")
    wid = s * 2 + c
    base = wid * CHUNK

    @pl.when(wid < NW - 1)
    def _full():
        _work(t_hbm, idx_hbm, out_hbm, idx_v, acc_v, sem, base, CHUNK, SUB)

    @pl.when(wid == NW - 1)
    def _last():
        _work(t_hbm, idx_hbm, out_hbm, idx_v, acc_v, sem, base, CHUNK_L, SUB_L)


_gather_sum = functools.partial(
    pl.kernel,
    out_type=jax.ShapeDtypeStruct((N_SITES, N_OUT), jnp.float32),
    mesh=plsc.VectorSubcoreMesh(core_axis_name="c", subcore_axis_name="s"),
    scratch_types=[
        pltpu.VMEM((FV * CHUNK,), jnp.int32),
        pltpu.VMEM((SUB, N_OUT), jnp.float32),
        pltpu.SemaphoreType.DMA,
    ],
)(_sc_body)


@jax.jit
def kernel(features, neighbor_idx, weight, bias):
    t = _transform(features, weight, bias)
    t_flat = t.reshape(FV * N_SITES, N_OUT)
    idx_t = (
        neighbor_idx.T
        + (jnp.arange(FV, dtype=jnp.int32) * N_SITES)[:, None]
    )
    return _gather_sum(t_flat, idx_t.reshape(FV * N_SITES))
